# baseline (speedup 1.0000x reference)
import jax
import jax.numpy as jnp
from jax.experimental import pallas as pl
from jax.experimental.pallas import tpu as pltpu

_BLK = 1024


def _vq_kernel(x_ref, w_ref, cb_ref, out_ref, wt_ref, cbt_ref, c2_ref):
    @pl.when(pl.program_id(0) == 0)
    def _():
        wt_ref[...] = w_ref[...].T
        cbt = cb_ref[...].T
        cbt_ref[...] = cbt
        c2_ref[...] = jnp.sum(cbt * cbt, axis=0, keepdims=True)

    h = jnp.dot(x_ref[...], wt_ref[...], preferred_element_type=jnp.float32)
    mean = jnp.mean(h, axis=-1, keepdims=True)
    hc = h - mean
    var = jnp.mean(hc * hc, axis=-1, keepdims=True)
    hn = hc * jax.lax.rsqrt(var + 1e-5)
    scores = jnp.dot(hn, cbt_ref[...], preferred_element_type=jnp.float32)
    val = c2_ref[...] - 2.0 * scores
    idx = jnp.argmin(val, axis=-1)
    out_ref[0, 0, :] = idx.astype(jnp.int32)


@jax.jit
def kernel(x, W, codebook):
    B, L, DIM = x.shape
    K, CD = codebook.shape
    N = B * L
    xf = x.reshape(N, DIM)
    grid = (N // _BLK,)
    out = pl.pallas_call(
        _vq_kernel,
        grid=grid,
        in_specs=[
            pl.BlockSpec((_BLK, DIM), lambda i: (i, 0)),
            pl.BlockSpec((CD, DIM), lambda i: (0, 0)),
            pl.BlockSpec((K, CD), lambda i: (0, 0)),
        ],
        out_specs=pl.BlockSpec((1, 1, _BLK), lambda i: (i, 0, 0)),
        out_shape=jax.ShapeDtypeStruct((N // _BLK, 1, _BLK), jnp.int32),
        scratch_shapes=[
            pltpu.VMEM((DIM, CD), jnp.float32),
            pltpu.VMEM((CD, K), jnp.float32),
            pltpu.VMEM((1, K), jnp.float32),
        ],
        compiler_params=pltpu.CompilerParams(
            dimension_semantics=("arbitrary",)),
    )(xf, W, codebook)
    return out.reshape(B, L)


# speedup vs baseline: 1.0033x; 1.0033x over previous
import jax
import jax.numpy as jnp
from jax.experimental import pallas as pl
from jax.experimental.pallas import tpu as pltpu

_BLK = 1024


def _vq_kernel(x_ref, w_ref, cb_ref, out_ref, wt_ref, cbt_ref, c2_ref):
    @pl.when(pl.program_id(0) == 0)
    def _():
        wt_ref[...] = w_ref[...].T
        cbt = cb_ref[...].T
        cbt_ref[...] = cbt
        c2_ref[...] = jnp.sum(cbt * cbt, axis=0, keepdims=True)

    h = jnp.dot(x_ref[...], wt_ref[...], preferred_element_type=jnp.float32)
    mean = jnp.mean(h, axis=-1, keepdims=True)
    hc = h - mean
    var = jnp.mean(hc * hc, axis=-1, keepdims=True)
    hn = hc * jax.lax.rsqrt(var + 1e-5)
    scores = jnp.dot(hn, cbt_ref[...], preferred_element_type=jnp.float32)
    val = c2_ref[...] - 2.0 * scores
    idx = jnp.argmin(val, axis=-1)
    out_ref[...] = idx.astype(jnp.int32)


@jax.jit
def kernel(x, W, codebook):
    B, L, DIM = x.shape
    K, CD = codebook.shape
    N = B * L
    xf = x.reshape(N, DIM)
    grid = (N // _BLK,)
    out = pl.pallas_call(
        _vq_kernel,
        grid=grid,
        in_specs=[
            pl.BlockSpec((_BLK, DIM), lambda i: (i, 0)),
            pl.BlockSpec((CD, DIM), lambda i: (0, 0)),
            pl.BlockSpec((K, CD), lambda i: (0, 0)),
        ],
        out_specs=pl.BlockSpec((_BLK,), lambda i: (i,)),
        out_shape=jax.ShapeDtypeStruct((N,), jnp.int32),
        scratch_shapes=[
            pltpu.VMEM((DIM, CD), jnp.float32),
            pltpu.VMEM((CD, K), jnp.float32),
            pltpu.VMEM((1, K), jnp.float32),
        ],
        compiler_params=pltpu.CompilerParams(
            dimension_semantics=("arbitrary",)),
    )(xf, W, codebook)
    return out.reshape(B, L)


# speedup vs baseline: 1.2333x; 1.2292x over previous
import jax
import jax.numpy as jnp
from jax.experimental import pallas as pl
from jax.experimental.pallas import tpu as pltpu

_BLK = 1024


def _vq_kernel(x_ref, w_ref, cb_ref, out_ref, wt_ref, c2_ref):
    @pl.when(pl.program_id(0) == 0)
    def _():
        wt_ref[...] = w_ref[...].T
        cb = cb_ref[...]
        c2_ref[...] = jnp.sum(cb * cb, axis=1, keepdims=True)

    h = jnp.dot(x_ref[...], wt_ref[...], preferred_element_type=jnp.float32)
    mean = jnp.mean(h, axis=-1, keepdims=True)
    hc = h - mean
    var = jnp.mean(hc * hc, axis=-1, keepdims=True)
    hn = hc * jax.lax.rsqrt(var + 1e-5)
    scores_t = jnp.dot(cb_ref[...], hn.T, preferred_element_type=jnp.float32)
    val = c2_ref[...] - 2.0 * scores_t
    idx = jnp.argmin(val, axis=0)
    out_ref[...] = idx.astype(jnp.int32)


@jax.jit
def kernel(x, W, codebook):
    B, L, DIM = x.shape
    K, CD = codebook.shape
    N = B * L
    xf = x.reshape(N, DIM)
    grid = (N // _BLK,)
    out = pl.pallas_call(
        _vq_kernel,
        grid=grid,
        in_specs=[
            pl.BlockSpec((_BLK, DIM), lambda i: (i, 0)),
            pl.BlockSpec((CD, DIM), lambda i: (0, 0)),
            pl.BlockSpec((K, CD), lambda i: (0, 0)),
        ],
        out_specs=pl.BlockSpec((_BLK,), lambda i: (i,)),
        out_shape=jax.ShapeDtypeStruct((N,), jnp.int32),
        scratch_shapes=[
            pltpu.VMEM((DIM, CD), jnp.float32),
            pltpu.VMEM((K, 1), jnp.float32),
        ],
        compiler_params=pltpu.CompilerParams(
            dimension_semantics=("arbitrary",)),
    )(xf, W, codebook)
    return out.reshape(B, L)


# speedup vs baseline: 1.2701x; 1.0298x over previous
import jax
import jax.numpy as jnp
from jax.experimental import pallas as pl
from jax.experimental.pallas import tpu as pltpu

_BLK = 2048


def _vq_kernel(x_ref, w_ref, cb_ref, out_ref, wt_ref, c2_ref):
    @pl.when(pl.program_id(0) == 0)
    def _():
        wt_ref[...] = w_ref[...].T
        cb = cb_ref[...]
        c2_ref[...] = jnp.sum(cb * cb, axis=1, keepdims=True)

    h = jnp.dot(x_ref[...], wt_ref[...], preferred_element_type=jnp.float32)
    mean = jnp.mean(h, axis=-1, keepdims=True)
    hc = h - mean
    var = jnp.mean(hc * hc, axis=-1, keepdims=True)
    hn = hc * jax.lax.rsqrt(var + 1e-5)
    scores_t = jnp.dot(cb_ref[...], hn.T, preferred_element_type=jnp.float32)
    val = c2_ref[...] - 2.0 * scores_t
    idx = jnp.argmin(val, axis=0)
    out_ref[...] = idx.astype(jnp.int32)


@jax.jit
def kernel(x, W, codebook):
    B, L, DIM = x.shape
    K, CD = codebook.shape
    N = B * L
    xf = x.reshape(N, DIM)
    grid = (N // _BLK,)
    out = pl.pallas_call(
        _vq_kernel,
        grid=grid,
        in_specs=[
            pl.BlockSpec((_BLK, DIM), lambda i: (i, 0)),
            pl.BlockSpec((CD, DIM), lambda i: (0, 0)),
            pl.BlockSpec((K, CD), lambda i: (0, 0)),
        ],
        out_specs=pl.BlockSpec((_BLK,), lambda i: (i,)),
        out_shape=jax.ShapeDtypeStruct((N,), jnp.int32),
        scratch_shapes=[
            pltpu.VMEM((DIM, CD), jnp.float32),
            pltpu.VMEM((K, 1), jnp.float32),
        ],
        compiler_params=pltpu.CompilerParams(
            dimension_semantics=("arbitrary",)),
    )(xf, W, codebook)
    return out.reshape(B, L)


# speedup vs baseline: 1.3714x; 1.0798x over previous
import jax
import jax.numpy as jnp
from jax.experimental import pallas as pl
from jax.experimental.pallas import tpu as pltpu

_BLK = 2048


def _vq_kernel(x_ref, w_ref, cbt_ref, out_ref, wt_ref, cb_ref, c2_ref):
    @pl.when(pl.program_id(0) == 0)
    def _():
        wt_ref[...] = w_ref[...].T
        cb = cbt_ref[...].T
        cb_ref[...] = cb
        c2_ref[...] = jnp.sum(cb * cb, axis=1, keepdims=True)

    h = jnp.dot(x_ref[...], wt_ref[...], preferred_element_type=jnp.float32)
    mean = jnp.mean(h, axis=-1, keepdims=True)
    hc = h - mean
    var = jnp.mean(hc * hc, axis=-1, keepdims=True)
    hn = hc * jax.lax.rsqrt(var + 1e-5)
    scores_t = jnp.dot(cb_ref[...], hn.T, preferred_element_type=jnp.float32)
    val = c2_ref[...] - 2.0 * scores_t
    idx = jnp.argmin(val, axis=0)
    out_ref[...] = idx.astype(jnp.int32)


@jax.jit
def kernel(x, W, codebook):
    B, L, DIM = x.shape
    K, CD = codebook.shape
    N = B * L
    xf = x.reshape(N, DIM)
    cbt = codebook.T
    grid = (N // _BLK,)
    out = pl.pallas_call(
        _vq_kernel,
        grid=grid,
        in_specs=[
            pl.BlockSpec((_BLK, DIM), lambda i: (i, 0)),
            pl.BlockSpec((CD, DIM), lambda i: (0, 0)),
            pl.BlockSpec((CD, K), lambda i: (0, 0)),
        ],
        out_specs=pl.BlockSpec((_BLK,), lambda i: (i,)),
        out_shape=jax.ShapeDtypeStruct((N,), jnp.int32),
        scratch_shapes=[
            pltpu.VMEM((DIM, CD), jnp.float32),
            pltpu.VMEM((K, CD), jnp.float32),
            pltpu.VMEM((K, 1), jnp.float32),
        ],
        compiler_params=pltpu.CompilerParams(
            dimension_semantics=("arbitrary",)),
    )(xf, W, cbt)
    return out.reshape(B, L)


# speedup vs baseline: 1.4479x; 1.0558x over previous
import jax
import jax.numpy as jnp
from jax.experimental import pallas as pl
from jax.experimental.pallas import tpu as pltpu

_BLK = 2048


def _vq_kernel(x_ref, w_ref, cbt_ref, out_ref, wt_ref, cb_ref, c2_ref):
    @pl.when(pl.program_id(0) == 0)
    def _():
        wt_ref[...] = w_ref[...].T
        cb = cbt_ref[...].T
        cb_ref[...] = cb
        c2_ref[...] = jnp.sum(cb * cb, axis=1, keepdims=True)

    H = x_ref.shape[0] // 2
    for p in range(2):
        h = jnp.dot(x_ref[p * H:(p + 1) * H, :], wt_ref[...],
                    preferred_element_type=jnp.float32)
        mean = jnp.mean(h, axis=-1, keepdims=True)
        hc = h - mean
        var = jnp.mean(hc * hc, axis=-1, keepdims=True)
        hn = hc * jax.lax.rsqrt(var + 1e-5)
        scores_t = jnp.dot(cb_ref[...], hn.T,
                           preferred_element_type=jnp.float32)
        val = c2_ref[...] - 2.0 * scores_t
        idx = jnp.argmin(val, axis=0)
        out_ref[p * H:(p + 1) * H] = idx.astype(jnp.int32)


@jax.jit
def kernel(x, W, codebook):
    B, L, DIM = x.shape
    K, CD = codebook.shape
    N = B * L
    xf = x.reshape(N, DIM)
    cbt = codebook.T
    grid = (N // _BLK,)
    out = pl.pallas_call(
        _vq_kernel,
        grid=grid,
        in_specs=[
            pl.BlockSpec((_BLK, DIM), lambda i: (i, 0)),
            pl.BlockSpec((CD, DIM), lambda i: (0, 0)),
            pl.BlockSpec((CD, K), lambda i: (0, 0)),
        ],
        out_specs=pl.BlockSpec((_BLK,), lambda i: (i,)),
        out_shape=jax.ShapeDtypeStruct((N,), jnp.int32),
        scratch_shapes=[
            pltpu.VMEM((DIM, CD), jnp.float32),
            pltpu.VMEM((K, CD), jnp.float32),
            pltpu.VMEM((K, 1), jnp.float32),
        ],
        compiler_params=pltpu.CompilerParams(
            dimension_semantics=("arbitrary",)),
    )(xf, W, cbt)
    return out.reshape(B, L)


# speedup vs baseline: 1.5303x; 1.0569x over previous
import jax
import jax.numpy as jnp
from jax.experimental import pallas as pl
from jax.experimental.pallas import tpu as pltpu

_BLK = 2048


def _vq_kernel(x_ref, w_ref, cbt_ref, out_ref, wt_ref, cb_ref, c2_ref):
    @pl.when(pl.program_id(0) == 0)
    def _():
        wt_ref[...] = w_ref[...].T
        cb = cbt_ref[...].T
        cb_ref[...] = cb
        c2_ref[...] = jnp.sum(cb * cb, axis=1, keepdims=True)

    H = x_ref.shape[0] // 4
    for p in range(4):
        h = jnp.dot(x_ref[p * H:(p + 1) * H, :], wt_ref[...],
                    preferred_element_type=jnp.float32)
        mean = jnp.mean(h, axis=-1, keepdims=True)
        hc = h - mean
        var = jnp.mean(hc * hc, axis=-1, keepdims=True)
        hn = hc * jax.lax.rsqrt(var + 1e-5)
        scores_t = jnp.dot(cb_ref[...], hn.T,
                           preferred_element_type=jnp.float32)
        val = c2_ref[...] - 2.0 * scores_t
        idx = jnp.argmin(val, axis=0)
        out_ref[p * H:(p + 1) * H] = idx.astype(jnp.int32)


@jax.jit
def kernel(x, W, codebook):
    B, L, DIM = x.shape
    K, CD = codebook.shape
    N = B * L
    xf = x.reshape(N, DIM)
    cbt = codebook.T
    grid = (N // _BLK,)
    out = pl.pallas_call(
        _vq_kernel,
        grid=grid,
        in_specs=[
            pl.BlockSpec((_BLK, DIM), lambda i: (i, 0)),
            pl.BlockSpec((CD, DIM), lambda i: (0, 0)),
            pl.BlockSpec((CD, K), lambda i: (0, 0)),
        ],
        out_specs=pl.BlockSpec((_BLK,), lambda i: (i,)),
        out_shape=jax.ShapeDtypeStruct((N,), jnp.int32),
        scratch_shapes=[
            pltpu.VMEM((DIM, CD), jnp.float32),
            pltpu.VMEM((K, CD), jnp.float32),
            pltpu.VMEM((K, 1), jnp.float32),
        ],
        compiler_params=pltpu.CompilerParams(
            dimension_semantics=("arbitrary",)),
    )(xf, W, cbt)
    return out.reshape(B, L)


# speedup vs baseline: 1.7152x; 1.1208x over previous
import jax
import jax.numpy as jnp
from jax.experimental import pallas as pl
from jax.experimental.pallas import tpu as pltpu

_BLK = 2048


def _vq_kernel(x_ref, w_ref, cbt_ref, out_ref, wt_ref, cb_ref, c2_ref):
    @pl.when(pl.program_id(0) == 0)
    def _():
        wt_ref[...] = w_ref[...].T
        cb = cbt_ref[...].T
        cb_ref[...] = cb
        c2_ref[...] = jnp.sum(cb * cb, axis=1, keepdims=True)

    H = x_ref.shape[0] // 8
    for p in range(8):
        h = jnp.dot(x_ref[p * H:(p + 1) * H, :], wt_ref[...],
                    preferred_element_type=jnp.float32)
        mean = jnp.mean(h, axis=-1, keepdims=True)
        hc = h - mean
        var = jnp.mean(hc * hc, axis=-1, keepdims=True)
        hn = hc * jax.lax.rsqrt(var + 1e-5)
        scores_t = jnp.dot(cb_ref[...], hn.T,
                           preferred_element_type=jnp.float32)
        val = c2_ref[...] - 2.0 * scores_t
        idx = jnp.argmin(val, axis=0)
        out_ref[p * H:(p + 1) * H] = idx.astype(jnp.int32)


@jax.jit
def kernel(x, W, codebook):
    B, L, DIM = x.shape
    K, CD = codebook.shape
    N = B * L
    xf = x.reshape(N, DIM)
    cbt = codebook.T
    grid = (N // _BLK,)
    out = pl.pallas_call(
        _vq_kernel,
        grid=grid,
        in_specs=[
            pl.BlockSpec((_BLK, DIM), lambda i: (i, 0)),
            pl.BlockSpec((CD, DIM), lambda i: (0, 0)),
            pl.BlockSpec((CD, K), lambda i: (0, 0)),
        ],
        out_specs=pl.BlockSpec((_BLK,), lambda i: (i,)),
        out_shape=jax.ShapeDtypeStruct((N,), jnp.int32),
        scratch_shapes=[
            pltpu.VMEM((DIM, CD), jnp.float32),
            pltpu.VMEM((K, CD), jnp.float32),
            pltpu.VMEM((K, 1), jnp.float32),
        ],
        compiler_params=pltpu.CompilerParams(
            dimension_semantics=("arbitrary",)),
    )(xf, W, cbt)
    return out.reshape(B, L)
